# baseline (speedup 1.0000x reference)
import functools

import jax
import jax.numpy as jnp
import numpy as np
from jax import lax
from jax.experimental import pallas as pl
from jax.experimental.pallas import tpu as pltpu
from jax.experimental.pallas import tpu_sc as plsc

_N = 2048
_D = 1024
_E = 8
_DFF = 1024
_LANE = 128
_NEG = -3.0e38
_EPS = float(np.finfo(np.float64).eps)

_B = 256
_NBLK = (2 * _N) // _B + _E
_CAP = _NBLK * _B
_NW = 32
_RPW = _CAP // _NW
_CHUNK = 48
_TPW = _N // _NW


def _gating_body(x_ref, wg_ref, g1_ref, g2_ref, pos_ref, blk_ref, ts_ref,
                 loss_ref, o_ref):
    x = x_ref[...]
    logits = jnp.dot(x, wg_ref[...], preferred_element_type=jnp.float32)
    n = logits.shape[0]
    col = jax.lax.broadcasted_iota(jnp.int32, (n, _LANE), 1)
    valid = col < _E
    neg = jnp.where(valid, logits, _NEG)
    m1 = jnp.max(neg, axis=1, keepdims=True)
    i1 = jnp.min(jnp.where(neg == m1, col, _LANE), axis=1, keepdims=True)
    neg2 = jnp.where(col == i1, _NEG, neg)
    m2 = jnp.max(neg2, axis=1, keepdims=True)
    i2 = jnp.min(jnp.where(neg2 == m2, col, _LANE), axis=1, keepdims=True)
    t = jnp.exp(m2 - m1)
    g1 = 1.0 / (1.0 + t)
    g2 = t / (1.0 + t)
    g1_ref[...] = g1
    g2_ref[...] = g2

    one_hot1 = jnp.where(col == i1, 1.0, 0.0)
    one_hot2 = jnp.where(col == i2, 1.0, 0.0)
    o_ref[:n, :] = one_hot1
    o_ref[n:, :] = one_hot2

    gates = one_hot1 * g1 + one_hot2 * g2
    imp = jnp.sum(gates, axis=0, keepdims=True)
    load = jnp.sum((gates > 0.0).astype(jnp.float32), axis=0, keepdims=True)

    def _cv2(v):
        mean = jnp.sum(jnp.where(col[:1] < _E, v, 0.0)) / _E
        var = jnp.sum(jnp.where(col[:1] < _E, (v - mean) ** 2, 0.0)) / (_E - 1)
        return var / (mean * mean + 1e-10)

    loss_ref[0, 0] = _cv2(imp) + _cv2(load)

    counts = jnp.sum(one_hot1, axis=0, keepdims=True) + jnp.sum(
        one_hot2, axis=0, keepdims=True)
    pblocks = jnp.floor((counts + (_B - 1.0)) / _B)
    rowi = jax.lax.broadcasted_iota(jnp.int32, (_LANE, _LANE), 0)
    coli = jax.lax.broadcasted_iota(jnp.int32, (_LANE, _LANE), 1)
    tri_excl = jnp.where(rowi < coli, 1.0, 0.0)
    tri_incl = jnp.where(rowi <= coli, 1.0, 0.0)
    starts = jnp.dot(pblocks * _B, tri_excl,
                     preferred_element_type=jnp.float32)
    ecum_b = jnp.dot(pblocks, tri_incl,
                     preferred_element_type=jnp.float32)

    sub = rowi.astype(jnp.float32)
    hit = jnp.where((jnp.broadcast_to(ecum_b, (_LANE, _LANE)) <= sub)
                    & (coli < _E), 1.0, 0.0)
    blk_ref[...] = jnp.minimum(jnp.sum(hit, axis=1, keepdims=True),
                               _E - 1.0).astype(jnp.int32)

    r256 = jax.lax.broadcasted_iota(jnp.int32, (_B, _B), 0)
    c256 = jax.lax.broadcasted_iota(jnp.int32, (_B, _B), 1)
    tri256 = jnp.where(r256 >= c256, 1.0, 0.0)

    def _chunk(j, carry):
        blk = o_ref[pl.ds(j * _B, _B), :]
        incl = jnp.dot(tri256, blk, preferred_element_type=jnp.float32)
        posm = incl - blk + carry + starts
        posr = jnp.sum(jnp.where(blk > 0.0, posm, 0.0), axis=1, keepdims=True)
        pos_ref[pl.ds(j * _B, _B), :] = posr.astype(jnp.int32)
        return carry + jnp.sum(blk, axis=0, keepdims=True)

    lax.fori_loop(0, (2 * n) // _B, _chunk, jnp.zeros((1, _LANE), jnp.float32))

    pos_all = pos_ref[...]
    arow = jax.lax.broadcasted_iota(jnp.int32, (1, 2 * n), 1)
    tokr = jnp.where(arow >= n, arow - n, arow)
    tok_hi = (tokr // 256).astype(jnp.float32)
    tok_lo = (tokr % 256).astype(jnp.float32)
    lane2n = jax.lax.broadcasted_iota(jnp.int32, (2 * n, _LANE), 1)

    def _tsblk(j, _):
        p_onehot = jnp.where(pos_all == j * _LANE + lane2n, 1.0, 0.0)
        hi = jnp.dot(tok_hi, p_onehot, preferred_element_type=jnp.float32)
        lo = jnp.dot(tok_lo, p_onehot, preferred_element_type=jnp.float32)
        ts_ref[pl.ds(j, 1), :] = (hi * 256.0 + lo).astype(jnp.int32)
        return 0

    lax.fori_loop(0, _CAP // _LANE, _tsblk, 0)


def _gating_call(x, wg_pad):
    n, d = x.shape
    return pl.pallas_call(
        _gating_body,
        out_shape=(
            jax.ShapeDtypeStruct((n, 1), jnp.float32),
            jax.ShapeDtypeStruct((n, 1), jnp.float32),
            jax.ShapeDtypeStruct((2 * n, 1), jnp.int32),
            jax.ShapeDtypeStruct((_LANE, 1), jnp.int32),
            jax.ShapeDtypeStruct((_CAP // _LANE, _LANE), jnp.int32),
            jax.ShapeDtypeStruct((1, 1), jnp.float32),
        ),
        in_specs=[
            pl.BlockSpec((n, d), lambda: (0, 0)),
            pl.BlockSpec((d, _LANE), lambda: (0, 0)),
        ],
        out_specs=(
            pl.BlockSpec((n, 1), lambda: (0, 0)),
            pl.BlockSpec((n, 1), lambda: (0, 0)),
            pl.BlockSpec((2 * n, 1), lambda: (0, 0)),
            pl.BlockSpec((_LANE, 1), lambda: (0, 0)),
            pl.BlockSpec((_CAP // _LANE, _LANE), lambda: (0, 0)),
            pl.BlockSpec(memory_space=pltpu.SMEM),
        ),
        scratch_shapes=[pltpu.VMEM((2 * n, _LANE), jnp.float32)],
    )(x, wg_pad)


def _dispatch_call(tok3d, x):
    mesh = plsc.VectorSubcoreMesh(core_axis_name="c", subcore_axis_name="s")
    nch = _RPW // _CHUNK

    @functools.partial(
        pl.kernel, mesh=mesh,
        out_type=jax.ShapeDtypeStruct((_CAP, _D), jnp.float32),
        scratch_types=[
            pltpu.VMEM((nch, _CHUNK), jnp.int32),
            pltpu.VMEM((2, _CHUNK, _D), jnp.float32),
            pltpu.SemaphoreType.DMA,
            pltpu.SemaphoreType.DMA,
        ],
    )
    def k(tok_hbm, x_hbm, xs_hbm, tok_v, rows_v, sem0, sem1):
        wid = lax.axis_index("c") * 16 + lax.axis_index("s")
        lo = wid * _RPW
        pltpu.sync_copy(tok_hbm.at[wid], tok_v)
        sems = (sem0, sem1)
        cur = pltpu.async_copy(x_hbm.at[tok_v.at[0]], rows_v.at[0], sem0)
        for j in range(nch):
            cur.wait()
            if j + 1 < nch:
                nxt = pltpu.async_copy(x_hbm.at[tok_v.at[j + 1]],
                                       rows_v.at[(j + 1) % 2],
                                       sems[(j + 1) % 2])
            pltpu.sync_copy(rows_v.at[j % 2],
                            xs_hbm.at[pl.ds(lo + j * _CHUNK, _CHUNK), :])
            if j + 1 < nch:
                cur = nxt

    return k(tok3d, x)


def _expert_body(be_ref, xs_ref, w1_ref, b1_ref, w2_ref, b2_ref, o_ref):
    h = jnp.dot(xs_ref[...], w1_ref[0],
                preferred_element_type=jnp.float32) + b1_ref[0]
    h = jnp.maximum(h, 0.0)
    out = jnp.dot(h, w2_ref[0], preferred_element_type=jnp.float32) + b2_ref[0]
    o_ref[...] = jnp.exp(out)


def _expert_call(x_sorted, blk_expert, W1, b1r, W2, b2r):
    grid_spec = pltpu.PrefetchScalarGridSpec(
        num_scalar_prefetch=1,
        grid=(_NBLK,),
        in_specs=[
            pl.BlockSpec((_B, _D), lambda i, be: (i, 0)),
            pl.BlockSpec((1, _D, _DFF), lambda i, be: (be[i], 0, 0)),
            pl.BlockSpec((1, 1, _DFF), lambda i, be: (be[i], 0, 0)),
            pl.BlockSpec((1, _DFF, _D), lambda i, be: (be[i], 0, 0)),
            pl.BlockSpec((1, 1, _D), lambda i, be: (be[i], 0, 0)),
        ],
        out_specs=pl.BlockSpec((_B, _D), lambda i, be: (i, 0)),
    )
    return pl.pallas_call(
        _expert_body,
        grid_spec=grid_spec,
        out_shape=jax.ShapeDtypeStruct((_CAP, _D), jnp.float32),
    )(blk_expert, x_sorted, W1, b1r, W2, b2r)


def _combine_gather_call(pos_r, out_exp):
    mesh = plsc.VectorSubcoreMesh(core_axis_name="c", subcore_axis_name="s")
    half = _TPW // 2

    @functools.partial(
        pl.kernel, mesh=mesh,
        out_type=(jax.ShapeDtypeStruct((_N, _D), jnp.float32),
                  jax.ShapeDtypeStruct((_N, _D), jnp.float32)),
        scratch_types=[
            pltpu.VMEM((2, 2, half), jnp.int32),
            pltpu.VMEM((2, half, _D), jnp.float32),
            pltpu.SemaphoreType.DMA,
            pltpu.SemaphoreType.DMA,
        ],
    )
    def k(pos_hbm, oe_hbm, c0_hbm, c1_hbm, idx_v, rows_v, sem0, sem1):
        wid = lax.axis_index("c") * 16 + lax.axis_index("s")
        t0 = wid * _TPW
        pltpu.sync_copy(pos_hbm.at[wid], idx_v)
        sems = (sem0, sem1)
        dsts = (c0_hbm, c0_hbm, c1_hbm, c1_hbm)
        cur = pltpu.async_copy(oe_hbm.at[idx_v.at[0, 0]], rows_v.at[0], sem0)
        for j in range(4):
            k_, h = j // 2, j % 2
            cur.wait()
            if j + 1 < 4:
                nxt = pltpu.async_copy(
                    oe_hbm.at[idx_v.at[(j + 1) // 2, (j + 1) % 2]],
                    rows_v.at[(j + 1) % 2], sems[(j + 1) % 2])
            pltpu.sync_copy(rows_v.at[j % 2],
                            dsts[j].at[pl.ds(t0 + h * half, half), :])
            if j + 1 < 4:
                cur = nxt

    return k(pos_r, out_exp)


def _combine_body(c0_ref, c1_ref, g1_ref, g2_ref, y_ref):
    bf = jnp.bfloat16
    g1 = g1_ref[...].astype(bf).astype(jnp.float32)
    g2 = g2_ref[...].astype(bf).astype(jnp.float32)
    c0 = c0_ref[...].astype(bf).astype(jnp.float32)
    c1 = c1_ref[...].astype(bf).astype(jnp.float32)
    comb = g1 * c0 + g2 * c1
    y_ref[...] = jnp.log(jnp.where(comb == 0.0, _EPS, comb))


def _combine_call(c0, c1, g1, g2):
    bn = 256
    return pl.pallas_call(
        _combine_body,
        grid=(_N // bn,),
        in_specs=[
            pl.BlockSpec((bn, _D), lambda i: (i, 0)),
            pl.BlockSpec((bn, _D), lambda i: (i, 0)),
            pl.BlockSpec((bn, 1), lambda i: (i, 0)),
            pl.BlockSpec((bn, 1), lambda i: (i, 0)),
        ],
        out_specs=pl.BlockSpec((bn, _D), lambda i: (i, 0)),
        out_shape=jax.ShapeDtypeStruct((_N, _D), jnp.float32),
    )(c0, c1, g1, g2)


def kernel(x, w_gate, w_noise, W1, b1, W2, b2):
    del w_noise
    wg_pad = jnp.pad(w_gate, ((0, 0), (0, _LANE - _E)))
    g1, g2, pos, blk_expert, tok_sorted, loss = _gating_call(x, wg_pad)
    pos_r = pos.reshape(2, _NW, 2, _TPW // 2).transpose(1, 0, 2, 3)
    blk = blk_expert.reshape(_LANE)[:_NBLK]
    tok3d = tok_sorted.reshape(_NW, _RPW // _CHUNK, _CHUNK)
    x_sorted = _dispatch_call(tok3d, x)
    out_exp = _expert_call(x_sorted, blk, W1, b1[:, None, :], W2,
                           b2[:, None, :])
    pos_km = pos.reshape(2 * _N)
    c0 = jnp.take(out_exp, pos_km[:_N], axis=0)
    c1 = jnp.take(out_exp, pos_km[_N:], axis=0)
    _ = pos_r
    y = _combine_call(c0, c1, g1, g2)
    return y, loss[0, 0]


# speedup vs baseline: 1.0528x; 1.0528x over previous
import functools

import jax
import jax.numpy as jnp
import numpy as np
from jax import lax
from jax.experimental import pallas as pl
from jax.experimental.pallas import tpu as pltpu
from jax.experimental.pallas import tpu_sc as plsc

_N = 2048
_D = 1024
_E = 8
_DFF = 1024
_LANE = 128
_NEG = -3.0e38
_EPS = float(np.finfo(np.float64).eps)

_B = 256
_NBLK = (2 * _N) // _B + _E
_CAP = _NBLK * _B
_NW = 32
_RPW = _CAP // _NW
_CHUNK = 48
_TPW = _N // _NW


def _gating_body(x_ref, wg_ref, g1_ref, g2_ref, pos_ref, blk_ref, ts_ref,
                 loss_ref, o_ref):
    x = x_ref[...]
    logits = jnp.dot(x, wg_ref[...], preferred_element_type=jnp.float32)
    n = logits.shape[0]
    col = jax.lax.broadcasted_iota(jnp.int32, (n, _LANE), 1)
    valid = col < _E
    neg = jnp.where(valid, logits, _NEG)
    m1 = jnp.max(neg, axis=1, keepdims=True)
    i1 = jnp.min(jnp.where(neg == m1, col, _LANE), axis=1, keepdims=True)
    neg2 = jnp.where(col == i1, _NEG, neg)
    m2 = jnp.max(neg2, axis=1, keepdims=True)
    i2 = jnp.min(jnp.where(neg2 == m2, col, _LANE), axis=1, keepdims=True)
    t = jnp.exp(m2 - m1)
    g1 = 1.0 / (1.0 + t)
    g2 = t / (1.0 + t)
    g1_ref[...] = g1
    g2_ref[...] = g2

    one_hot1 = jnp.where(col == i1, 1.0, 0.0)
    one_hot2 = jnp.where(col == i2, 1.0, 0.0)
    o_ref[:n, :] = one_hot1
    o_ref[n:, :] = one_hot2

    gates = one_hot1 * g1 + one_hot2 * g2
    imp = jnp.sum(gates, axis=0, keepdims=True)
    load = jnp.sum((gates > 0.0).astype(jnp.float32), axis=0, keepdims=True)

    def _cv2(v):
        mean = jnp.sum(jnp.where(col[:1] < _E, v, 0.0)) / _E
        var = jnp.sum(jnp.where(col[:1] < _E, (v - mean) ** 2, 0.0)) / (_E - 1)
        return var / (mean * mean + 1e-10)

    loss_ref[0, 0] = _cv2(imp) + _cv2(load)

    counts = jnp.sum(one_hot1, axis=0, keepdims=True) + jnp.sum(
        one_hot2, axis=0, keepdims=True)
    pblocks = jnp.floor((counts + (_B - 1.0)) / _B)
    rowi = jax.lax.broadcasted_iota(jnp.int32, (_LANE, _LANE), 0)
    coli = jax.lax.broadcasted_iota(jnp.int32, (_LANE, _LANE), 1)
    tri_excl = jnp.where(rowi < coli, 1.0, 0.0)
    tri_incl = jnp.where(rowi <= coli, 1.0, 0.0)
    starts = jnp.dot(pblocks * _B, tri_excl,
                     preferred_element_type=jnp.float32)
    ecum_b = jnp.dot(pblocks, tri_incl,
                     preferred_element_type=jnp.float32)

    sub = rowi.astype(jnp.float32)
    hit = jnp.where((jnp.broadcast_to(ecum_b, (_LANE, _LANE)) <= sub)
                    & (coli < _E), 1.0, 0.0)
    blk_ref[...] = jnp.minimum(jnp.sum(hit, axis=1, keepdims=True),
                               _E - 1.0).astype(jnp.int32)

    r256 = jax.lax.broadcasted_iota(jnp.int32, (_B, _B), 0)
    c256 = jax.lax.broadcasted_iota(jnp.int32, (_B, _B), 1)
    tri256 = jnp.where(r256 >= c256, 1.0, 0.0)

    def _chunk(j, carry):
        blk = o_ref[pl.ds(j * _B, _B), :]
        incl = jnp.dot(tri256, blk, preferred_element_type=jnp.float32)
        posm = incl - blk + carry + starts
        posr = jnp.sum(jnp.where(blk > 0.0, posm, 0.0), axis=1, keepdims=True)
        pos_ref[pl.ds(j * _B, _B), :] = posr.astype(jnp.int32)
        return carry + jnp.sum(blk, axis=0, keepdims=True)

    lax.fori_loop(0, (2 * n) // _B, _chunk, jnp.zeros((1, _LANE), jnp.float32))

    pos_all = pos_ref[...]
    arow = jax.lax.broadcasted_iota(jnp.int32, (1, 2 * n), 1)
    tokr = jnp.where(arow >= n, arow - n, arow)
    tok_hi = (tokr // 256).astype(jnp.float32)
    tok_lo = (tokr % 256).astype(jnp.float32)
    lane2n = jax.lax.broadcasted_iota(jnp.int32, (2 * n, _LANE), 1)

    def _tsblk(j, _):
        p_onehot = jnp.where(pos_all == j * _LANE + lane2n, 1.0, 0.0)
        hi = jnp.dot(tok_hi, p_onehot, preferred_element_type=jnp.float32)
        lo = jnp.dot(tok_lo, p_onehot, preferred_element_type=jnp.float32)
        ts_ref[pl.ds(j, 1), :] = (hi * 256.0 + lo).astype(jnp.int32)
        return 0

    lax.fori_loop(0, _CAP // _LANE, _tsblk, 0)


def _gating_call(x, wg_pad):
    n, d = x.shape
    return pl.pallas_call(
        _gating_body,
        out_shape=(
            jax.ShapeDtypeStruct((n, 1), jnp.float32),
            jax.ShapeDtypeStruct((n, 1), jnp.float32),
            jax.ShapeDtypeStruct((2 * n, 1), jnp.int32),
            jax.ShapeDtypeStruct((_LANE, 1), jnp.int32),
            jax.ShapeDtypeStruct((_CAP // _LANE, _LANE), jnp.int32),
            jax.ShapeDtypeStruct((1, 1), jnp.float32),
        ),
        in_specs=[
            pl.BlockSpec((n, d), lambda: (0, 0)),
            pl.BlockSpec((d, _LANE), lambda: (0, 0)),
        ],
        out_specs=(
            pl.BlockSpec((n, 1), lambda: (0, 0)),
            pl.BlockSpec((n, 1), lambda: (0, 0)),
            pl.BlockSpec((2 * n, 1), lambda: (0, 0)),
            pl.BlockSpec((_LANE, 1), lambda: (0, 0)),
            pl.BlockSpec((_CAP // _LANE, _LANE), lambda: (0, 0)),
            pl.BlockSpec(memory_space=pltpu.SMEM),
        ),
        scratch_shapes=[pltpu.VMEM((2 * n, _LANE), jnp.float32)],
    )(x, wg_pad)


def _dispatch_call(tok3d, x):
    mesh = plsc.VectorSubcoreMesh(core_axis_name="c", subcore_axis_name="s")
    nch = _RPW // _CHUNK

    @functools.partial(
        pl.kernel, mesh=mesh,
        out_type=jax.ShapeDtypeStruct((_CAP, _D), jnp.float32),
        scratch_types=[
            pltpu.VMEM((nch, _CHUNK), jnp.int32),
            pltpu.VMEM((2, _CHUNK, _D), jnp.float32),
            pltpu.SemaphoreType.DMA,
            pltpu.SemaphoreType.DMA,
        ],
    )
    def k(tok_hbm, x_hbm, xs_hbm, tok_v, rows_v, sem0, sem1):
        wid = lax.axis_index("c") * 16 + lax.axis_index("s")
        lo = wid * _RPW
        pltpu.sync_copy(tok_hbm.at[wid], tok_v)
        sems = (sem0, sem1)
        cur = pltpu.async_copy(x_hbm.at[tok_v.at[0]], rows_v.at[0], sem0)
        for j in range(nch):
            cur.wait()
            if j + 1 < nch:
                nxt = pltpu.async_copy(x_hbm.at[tok_v.at[j + 1]],
                                       rows_v.at[(j + 1) % 2],
                                       sems[(j + 1) % 2])
            pltpu.sync_copy(rows_v.at[j % 2],
                            xs_hbm.at[pl.ds(lo + j * _CHUNK, _CHUNK), :])
            if j + 1 < nch:
                cur = nxt

    return k(tok3d, x)


def _expert_body(be_ref, xs_ref, w1_ref, b1_ref, w2_ref, b2_ref, o_ref):
    h = jnp.dot(xs_ref[...], w1_ref[0],
                preferred_element_type=jnp.float32) + b1_ref[0]
    h = jnp.maximum(h, 0.0)
    out = jnp.dot(h, w2_ref[0], preferred_element_type=jnp.float32) + b2_ref[0]
    o_ref[...] = jnp.exp(out)


def _expert_call(x_sorted, blk_expert, W1, b1r, W2, b2r):
    grid_spec = pltpu.PrefetchScalarGridSpec(
        num_scalar_prefetch=1,
        grid=(_NBLK,),
        in_specs=[
            pl.BlockSpec((_B, _D), lambda i, be: (i, 0)),
            pl.BlockSpec((1, _D, _DFF), lambda i, be: (be[i], 0, 0)),
            pl.BlockSpec((1, 1, _DFF), lambda i, be: (be[i], 0, 0)),
            pl.BlockSpec((1, _DFF, _D), lambda i, be: (be[i], 0, 0)),
            pl.BlockSpec((1, 1, _D), lambda i, be: (be[i], 0, 0)),
        ],
        out_specs=pl.BlockSpec((_B, _D), lambda i, be: (i, 0)),
    )
    return pl.pallas_call(
        _expert_body,
        grid_spec=grid_spec,
        out_shape=jax.ShapeDtypeStruct((_CAP, _D), jnp.float32),
    )(blk_expert, x_sorted, W1, b1r, W2, b2r)


def _combine_gather_call(pos_r, out_exp):
    mesh = plsc.VectorSubcoreMesh(core_axis_name="c", subcore_axis_name="s")
    half = _TPW // 2

    @functools.partial(
        pl.kernel, mesh=mesh,
        out_type=(jax.ShapeDtypeStruct((_N, _D), jnp.float32),
                  jax.ShapeDtypeStruct((_N, _D), jnp.float32)),
        scratch_types=[
            pltpu.VMEM((2, 2, half), jnp.int32),
            pltpu.VMEM((2, half, _D), jnp.float32),
            pltpu.SemaphoreType.DMA,
            pltpu.SemaphoreType.DMA,
        ],
    )
    def k(pos_hbm, oe_hbm, c0_hbm, c1_hbm, idx_v, rows_v, sem0, sem1):
        wid = lax.axis_index("c") * 16 + lax.axis_index("s")
        t0 = wid * _TPW
        pltpu.sync_copy(pos_hbm.at[wid], idx_v)
        sems = (sem0, sem1)
        dsts = (c0_hbm, c0_hbm, c1_hbm, c1_hbm)
        cur = pltpu.async_copy(oe_hbm.at[idx_v.at[0, 0]], rows_v.at[0], sem0)
        for j in range(4):
            k_, h = j // 2, j % 2
            cur.wait()
            if j + 1 < 4:
                nxt = pltpu.async_copy(
                    oe_hbm.at[idx_v.at[(j + 1) // 2, (j + 1) % 2]],
                    rows_v.at[(j + 1) % 2], sems[(j + 1) % 2])
            pltpu.sync_copy(rows_v.at[j % 2],
                            dsts[j].at[pl.ds(t0 + h * half, half), :])
            if j + 1 < 4:
                cur = nxt

    return k(pos_r, out_exp)


def _combine_body(c0_ref, c1_ref, g1_ref, g2_ref, y_ref):
    bf = jnp.bfloat16
    g1 = g1_ref[...].astype(bf).astype(jnp.float32)
    g2 = g2_ref[...].astype(bf).astype(jnp.float32)
    c0 = c0_ref[...].astype(bf).astype(jnp.float32)
    c1 = c1_ref[...].astype(bf).astype(jnp.float32)
    comb = g1 * c0 + g2 * c1
    y_ref[...] = jnp.log(jnp.where(comb == 0.0, _EPS, comb))


def _combine_call(c0, c1, g1, g2):
    bn = 256
    return pl.pallas_call(
        _combine_body,
        grid=(_N // bn,),
        in_specs=[
            pl.BlockSpec((bn, _D), lambda i: (i, 0)),
            pl.BlockSpec((bn, _D), lambda i: (i, 0)),
            pl.BlockSpec((bn, 1), lambda i: (i, 0)),
            pl.BlockSpec((bn, 1), lambda i: (i, 0)),
        ],
        out_specs=pl.BlockSpec((bn, _D), lambda i: (i, 0)),
        out_shape=jax.ShapeDtypeStruct((_N, _D), jnp.float32),
    )(c0, c1, g1, g2)


def kernel(x, w_gate, w_noise, W1, b1, W2, b2):
    del w_noise
    wg_pad = jnp.pad(w_gate, ((0, 0), (0, _LANE - _E)))
    g1, g2, pos, blk_expert, tok_sorted, loss = _gating_call(x, wg_pad)
    pos_r = pos.reshape(2, _NW, 2, _TPW // 2).transpose(1, 0, 2, 3)
    blk = blk_expert.reshape(_LANE)[:_NBLK]
    tok3d = tok_sorted.reshape(_NW, _RPW // _CHUNK, _CHUNK)
    x_c = jax.lax.optimization_barrier(x * 1.0)
    x_sorted = _dispatch_call(tok3d, x_c)
    out_exp = _expert_call(x_sorted, blk, W1, b1[:, None, :], W2,
                           b2[:, None, :])
    c0, c1 = _combine_gather_call(pos_r, out_exp)
    y = _combine_call(c0, c1, g1, g2)
    return y, loss[0, 0]


# speedup vs baseline: 1.6801x; 1.5959x over previous
import functools

import jax
import jax.numpy as jnp
import numpy as np
from jax import lax
from jax.experimental import pallas as pl
from jax.experimental.pallas import tpu as pltpu
from jax.experimental.pallas import tpu_sc as plsc

_N = 2048
_D = 1024
_E = 8
_DFF = 1024
_LANE = 128
_NEG = -3.0e38
_EPS = float(np.finfo(np.float64).eps)

_B = 256
_NBLK = (2 * _N) // _B + _E
_CAP = _NBLK * _B
_NW = 32
_RPW = _CAP // _NW
_CHUNK = 48
_TPW = _N // _NW


def _gating_body(x_ref, wg_ref, g1_ref, g2_ref, pos_ref, blk_ref, ts_ref,
                 loss_ref, o_ref):
    x = x_ref[...]
    logits = jnp.dot(x, wg_ref[...], preferred_element_type=jnp.float32)
    n = logits.shape[0]
    col = jax.lax.broadcasted_iota(jnp.int32, (n, _LANE), 1)
    valid = col < _E
    neg = jnp.where(valid, logits, _NEG)
    m1 = jnp.max(neg, axis=1, keepdims=True)
    i1 = jnp.min(jnp.where(neg == m1, col, _LANE), axis=1, keepdims=True)
    neg2 = jnp.where(col == i1, _NEG, neg)
    m2 = jnp.max(neg2, axis=1, keepdims=True)
    i2 = jnp.min(jnp.where(neg2 == m2, col, _LANE), axis=1, keepdims=True)
    t = jnp.exp(m2 - m1)
    g1 = 1.0 / (1.0 + t)
    g2 = t / (1.0 + t)
    g1_ref[...] = g1
    g2_ref[...] = g2

    one_hot1 = jnp.where(col == i1, 1.0, 0.0)
    one_hot2 = jnp.where(col == i2, 1.0, 0.0)
    o_ref[:n, :] = one_hot1
    o_ref[n:, :] = one_hot2

    gates = one_hot1 * g1 + one_hot2 * g2
    imp = jnp.sum(gates, axis=0, keepdims=True)
    load = jnp.sum((gates > 0.0).astype(jnp.float32), axis=0, keepdims=True)

    def _cv2(v):
        mean = jnp.sum(jnp.where(col[:1] < _E, v, 0.0)) / _E
        var = jnp.sum(jnp.where(col[:1] < _E, (v - mean) ** 2, 0.0)) / (_E - 1)
        return var / (mean * mean + 1e-10)

    loss_ref[0, 0] = _cv2(imp) + _cv2(load)

    counts = jnp.sum(one_hot1, axis=0, keepdims=True) + jnp.sum(
        one_hot2, axis=0, keepdims=True)
    pblocks = jnp.floor((counts + (_B - 1.0)) / _B)
    rowi = jax.lax.broadcasted_iota(jnp.int32, (_LANE, _LANE), 0)
    coli = jax.lax.broadcasted_iota(jnp.int32, (_LANE, _LANE), 1)
    tri_excl = jnp.where(rowi < coli, 1.0, 0.0)
    tri_incl = jnp.where(rowi <= coli, 1.0, 0.0)
    starts = jnp.dot(pblocks * _B, tri_excl,
                     preferred_element_type=jnp.float32)
    ecum_b = jnp.dot(pblocks, tri_incl,
                     preferred_element_type=jnp.float32)

    sub = rowi.astype(jnp.float32)
    hit = jnp.where((jnp.broadcast_to(ecum_b, (_LANE, _LANE)) <= sub)
                    & (coli < _E), 1.0, 0.0)
    blk_ref[...] = jnp.minimum(jnp.sum(hit, axis=1, keepdims=True),
                               _E - 1.0).astype(jnp.int32)

    r256 = jax.lax.broadcasted_iota(jnp.int32, (_B, _B), 0)
    c256 = jax.lax.broadcasted_iota(jnp.int32, (_B, _B), 1)
    tri256 = jnp.where(r256 >= c256, 1.0, 0.0)

    def _chunk(j, carry):
        blk = o_ref[pl.ds(j * _B, _B), :]
        incl = jnp.dot(tri256, blk, preferred_element_type=jnp.float32)
        posm = incl - blk + carry + starts
        posr = jnp.sum(jnp.where(blk > 0.0, posm, 0.0), axis=1, keepdims=True)
        pos_ref[pl.ds(j * _B, _B), :] = posr.astype(jnp.int32)
        return carry + jnp.sum(blk, axis=0, keepdims=True)

    lax.fori_loop(0, (2 * n) // _B, _chunk, jnp.zeros((1, _LANE), jnp.float32))

    pos_all = pos_ref[...]
    arow = jax.lax.broadcasted_iota(jnp.int32, (1, 2 * n), 1)
    tokr = jnp.where(arow >= n, arow - n, arow)
    hls = jnp.concatenate(
        [(tokr // 256).astype(jnp.float32),
         (tokr % 256).astype(jnp.float32),
         jnp.ones((1, 2 * n), jnp.float32)], axis=0)
    lane2n = jax.lax.broadcasted_iota(jnp.int32, (2 * n, _LANE), 1)
    lane1 = jax.lax.broadcasted_iota(jnp.int32, (1, _LANE), 1)

    def _tsblk(j, _):
        p_onehot = jnp.where(pos_all == j * _LANE + lane2n, 1.0, 0.0)
        hlc = jnp.dot(hls, p_onehot, preferred_element_type=jnp.float32)
        tok = hlc[0:1] * 256.0 + hlc[1:2]
        v = j * _LANE + lane1
        v = jnp.where(v >= 2 * n, v - 2 * n, jnp.where(v >= n, v - n, v))
        ts_ref[pl.ds(j, 1), :] = jnp.where(
            hlc[2:3] > 0.0, tok, v.astype(jnp.float32)).astype(jnp.int32)
        return 0

    lax.fori_loop(0, _CAP // _LANE, _tsblk, 0)


def _gating_call(x, wg_pad):
    n, d = x.shape
    return pl.pallas_call(
        _gating_body,
        out_shape=(
            jax.ShapeDtypeStruct((n, 1), jnp.float32),
            jax.ShapeDtypeStruct((n, 1), jnp.float32),
            jax.ShapeDtypeStruct((2 * n, 1), jnp.int32),
            jax.ShapeDtypeStruct((_LANE, 1), jnp.int32),
            jax.ShapeDtypeStruct((_CAP // _LANE, _LANE), jnp.int32),
            jax.ShapeDtypeStruct((1, 1), jnp.float32),
        ),
        in_specs=[
            pl.BlockSpec((n, d), lambda: (0, 0)),
            pl.BlockSpec((d, _LANE), lambda: (0, 0)),
        ],
        out_specs=(
            pl.BlockSpec((n, 1), lambda: (0, 0)),
            pl.BlockSpec((n, 1), lambda: (0, 0)),
            pl.BlockSpec((2 * n, 1), lambda: (0, 0)),
            pl.BlockSpec((_LANE, 1), lambda: (0, 0)),
            pl.BlockSpec((_CAP // _LANE, _LANE), lambda: (0, 0)),
            pl.BlockSpec(memory_space=pltpu.SMEM),
        ),
        scratch_shapes=[pltpu.VMEM((2 * n, _LANE), jnp.float32)],
    )(x, wg_pad)


def _dispatch_call(tok3d, x):
    mesh = plsc.VectorSubcoreMesh(core_axis_name="c", subcore_axis_name="s")
    nch = _RPW // _CHUNK

    @functools.partial(
        pl.kernel, mesh=mesh,
        out_type=jax.ShapeDtypeStruct((_CAP, _D), jnp.float32),
        scratch_types=[
            pltpu.VMEM((nch, _CHUNK), jnp.int32),
            pltpu.VMEM((2, _CHUNK, _D), jnp.float32),
            pltpu.SemaphoreType.DMA,
            pltpu.SemaphoreType.DMA,
        ],
    )
    def k(tok_hbm, x_hbm, xs_hbm, tok_v, rows_v, sem0, sem1):
        wid = lax.axis_index("c") * 16 + lax.axis_index("s")
        lo = wid * _RPW
        pltpu.sync_copy(tok_hbm.at[wid], tok_v)
        sems = (sem0, sem1)
        cur = pltpu.async_copy(x_hbm.at[tok_v.at[0]], rows_v.at[0], sem0)
        for j in range(nch):
            cur.wait()
            if j + 1 < nch:
                nxt = pltpu.async_copy(x_hbm.at[tok_v.at[j + 1]],
                                       rows_v.at[(j + 1) % 2],
                                       sems[(j + 1) % 2])
            pltpu.sync_copy(rows_v.at[j % 2],
                            xs_hbm.at[pl.ds(lo + j * _CHUNK, _CHUNK), :])
            if j + 1 < nch:
                cur = nxt

    return k(tok3d, x)


def _expert_body(be_ref, xs_ref, w1_ref, b1_ref, w2_ref, b2_ref, o_ref):
    h = jnp.dot(xs_ref[...], w1_ref[0],
                preferred_element_type=jnp.float32) + b1_ref[0]
    h = jnp.maximum(h, 0.0)
    out = jnp.dot(h, w2_ref[0], preferred_element_type=jnp.float32) + b2_ref[0]
    o_ref[...] = jnp.exp(out)


def _expert_call(x_sorted, blk_expert, W1, b1r, W2, b2r):
    grid_spec = pltpu.PrefetchScalarGridSpec(
        num_scalar_prefetch=1,
        grid=(_NBLK,),
        in_specs=[
            pl.BlockSpec((_B, _D), lambda i, be: (i, 0)),
            pl.BlockSpec((1, _D, _DFF), lambda i, be: (be[i], 0, 0)),
            pl.BlockSpec((1, 1, _DFF), lambda i, be: (be[i], 0, 0)),
            pl.BlockSpec((1, _DFF, _D), lambda i, be: (be[i], 0, 0)),
            pl.BlockSpec((1, 1, _D), lambda i, be: (be[i], 0, 0)),
        ],
        out_specs=pl.BlockSpec((_B, _D), lambda i, be: (i, 0)),
    )
    return pl.pallas_call(
        _expert_body,
        grid_spec=grid_spec,
        out_shape=jax.ShapeDtypeStruct((_CAP, _D), jnp.float32),
    )(blk_expert, x_sorted, W1, b1r, W2, b2r)


def _combine_gather_call(pos_r, out_exp):
    mesh = plsc.VectorSubcoreMesh(core_axis_name="c", subcore_axis_name="s")
    half = _TPW // 2

    @functools.partial(
        pl.kernel, mesh=mesh,
        out_type=(jax.ShapeDtypeStruct((_N, _D), jnp.float32),
                  jax.ShapeDtypeStruct((_N, _D), jnp.float32)),
        scratch_types=[
            pltpu.VMEM((2, 2, half), jnp.int32),
            pltpu.VMEM((2, half, _D), jnp.float32),
            pltpu.SemaphoreType.DMA,
            pltpu.SemaphoreType.DMA,
        ],
    )
    def k(pos_hbm, oe_hbm, c0_hbm, c1_hbm, idx_v, rows_v, sem0, sem1):
        wid = lax.axis_index("c") * 16 + lax.axis_index("s")
        t0 = wid * _TPW
        pltpu.sync_copy(pos_hbm.at[wid], idx_v)
        sems = (sem0, sem1)
        dsts = (c0_hbm, c0_hbm, c1_hbm, c1_hbm)
        cur = pltpu.async_copy(oe_hbm.at[idx_v.at[0, 0]], rows_v.at[0], sem0)
        for j in range(4):
            k_, h = j // 2, j % 2
            cur.wait()
            if j + 1 < 4:
                nxt = pltpu.async_copy(
                    oe_hbm.at[idx_v.at[(j + 1) // 2, (j + 1) % 2]],
                    rows_v.at[(j + 1) % 2], sems[(j + 1) % 2])
            pltpu.sync_copy(rows_v.at[j % 2],
                            dsts[j].at[pl.ds(t0 + h * half, half), :])
            if j + 1 < 4:
                cur = nxt

    return k(pos_r, out_exp)


def _combine_body(c0_ref, c1_ref, g1_ref, g2_ref, y_ref):
    bf = jnp.bfloat16
    g1 = g1_ref[...].astype(bf).astype(jnp.float32)
    g2 = g2_ref[...].astype(bf).astype(jnp.float32)
    c0 = c0_ref[...].astype(bf).astype(jnp.float32)
    c1 = c1_ref[...].astype(bf).astype(jnp.float32)
    comb = g1 * c0 + g2 * c1
    y_ref[...] = jnp.log(jnp.where(comb == 0.0, _EPS, comb))


def _combine_call(c0, c1, g1, g2):
    bn = 256
    return pl.pallas_call(
        _combine_body,
        grid=(_N // bn,),
        in_specs=[
            pl.BlockSpec((bn, _D), lambda i: (i, 0)),
            pl.BlockSpec((bn, _D), lambda i: (i, 0)),
            pl.BlockSpec((bn, 1), lambda i: (i, 0)),
            pl.BlockSpec((bn, 1), lambda i: (i, 0)),
        ],
        out_specs=pl.BlockSpec((bn, _D), lambda i: (i, 0)),
        out_shape=jax.ShapeDtypeStruct((_N, _D), jnp.float32),
    )(c0, c1, g1, g2)


def kernel(x, w_gate, w_noise, W1, b1, W2, b2):
    del w_noise
    wg_pad = jnp.pad(w_gate, ((0, 0), (0, _LANE - _E)))
    g1, g2, pos, blk_expert, tok_sorted, loss = _gating_call(x, wg_pad)
    pos_r = pos.reshape(2, _NW, 2, _TPW // 2).transpose(1, 0, 2, 3)
    blk = blk_expert.reshape(_LANE)[:_NBLK]
    tok3d = tok_sorted.reshape(_NW, _RPW // _CHUNK, _CHUNK)
    x_sorted = _dispatch_call(tok3d, x)
    out_exp = _expert_call(x_sorted, blk, W1, b1[:, None, :], W2,
                           b2[:, None, :])
    c0, c1 = _combine_gather_call(pos_r, out_exp)
    y = _combine_call(c0, c1, g1, g2)
    return y, loss[0, 0]


# speedup vs baseline: 2.2318x; 1.3284x over previous
import functools

import jax
import jax.numpy as jnp
import numpy as np
from jax import lax
from jax.experimental import pallas as pl
from jax.experimental.pallas import tpu as pltpu
from jax.experimental.pallas import tpu_sc as plsc

_N = 2048
_D = 1024
_E = 8
_DFF = 1024
_LANE = 128
_NEG = -3.0e38
_EPS = float(np.finfo(np.float64).eps)

_B = 256
_NBLK = (2 * _N) // _B + _E
_CAP = _NBLK * _B
_NW = 32
_RPW = _CAP // _NW
_CHUNK = 48
_TPW = _N // _NW


def _gating_body(x_ref, wg_ref, g1_ref, g2_ref, pos_ref, blk_ref,
                 loss_ref, o_ref):
    x = x_ref[...]
    logits = jnp.dot(x, wg_ref[...], preferred_element_type=jnp.float32)
    n = logits.shape[0]
    col = jax.lax.broadcasted_iota(jnp.int32, (n, _LANE), 1)
    valid = col < _E
    neg = jnp.where(valid, logits, _NEG)
    m1 = jnp.max(neg, axis=1, keepdims=True)
    i1 = jnp.min(jnp.where(neg == m1, col, _LANE), axis=1, keepdims=True)
    neg2 = jnp.where(col == i1, _NEG, neg)
    m2 = jnp.max(neg2, axis=1, keepdims=True)
    i2 = jnp.min(jnp.where(neg2 == m2, col, _LANE), axis=1, keepdims=True)
    t = jnp.exp(m2 - m1)
    g1 = 1.0 / (1.0 + t)
    g2 = t / (1.0 + t)
    g1_ref[...] = g1
    g2_ref[...] = g2

    one_hot1 = jnp.where(col == i1, 1.0, 0.0)
    one_hot2 = jnp.where(col == i2, 1.0, 0.0)
    o_ref[:n, :] = one_hot1
    o_ref[n:, :] = one_hot2

    gates = one_hot1 * g1 + one_hot2 * g2
    imp = jnp.sum(gates, axis=0, keepdims=True)
    load = jnp.sum((gates > 0.0).astype(jnp.float32), axis=0, keepdims=True)

    def _cv2(v):
        mean = jnp.sum(jnp.where(col[:1] < _E, v, 0.0)) / _E
        var = jnp.sum(jnp.where(col[:1] < _E, (v - mean) ** 2, 0.0)) / (_E - 1)
        return var / (mean * mean + 1e-10)

    loss_ref[0, 0] = _cv2(imp) + _cv2(load)

    counts = jnp.sum(one_hot1, axis=0, keepdims=True) + jnp.sum(
        one_hot2, axis=0, keepdims=True)
    pblocks = jnp.floor((counts + (_B - 1.0)) / _B)
    rowi = jax.lax.broadcasted_iota(jnp.int32, (_LANE, _LANE), 0)
    coli = jax.lax.broadcasted_iota(jnp.int32, (_LANE, _LANE), 1)
    tri_excl = jnp.where(rowi < coli, 1.0, 0.0)
    tri_incl = jnp.where(rowi <= coli, 1.0, 0.0)
    starts = jnp.dot(pblocks * _B, tri_excl,
                     preferred_element_type=jnp.float32)
    ecum_b = jnp.dot(pblocks, tri_incl,
                     preferred_element_type=jnp.float32)

    sub = rowi.astype(jnp.float32)
    hit = jnp.where((jnp.broadcast_to(ecum_b, (_LANE, _LANE)) <= sub)
                    & (coli < _E), 1.0, 0.0)
    blk_ref[...] = jnp.minimum(jnp.sum(hit, axis=1, keepdims=True),
                               _E - 1.0).astype(jnp.int32)

    r256 = jax.lax.broadcasted_iota(jnp.int32, (_B, _B), 0)
    c256 = jax.lax.broadcasted_iota(jnp.int32, (_B, _B), 1)
    tri256 = jnp.where(r256 >= c256, 1.0, 0.0)

    def _chunk(j, carry):
        blk = o_ref[pl.ds(j * _B, _B), :]
        incl = jnp.dot(tri256, blk, preferred_element_type=jnp.float32)
        posm = incl - blk + carry + starts
        posr = jnp.sum(jnp.where(blk > 0.0, posm, 0.0), axis=1, keepdims=True)
        pos_ref[pl.ds(j * _B, _B), :] = posr.astype(jnp.int32)
        return carry + jnp.sum(blk, axis=0, keepdims=True)

    lax.fori_loop(0, (2 * n) // _B, _chunk, jnp.zeros((1, _LANE), jnp.float32))


def _gating_call(x, wg_pad):
    n, d = x.shape
    return pl.pallas_call(
        _gating_body,
        out_shape=(
            jax.ShapeDtypeStruct((n, 1), jnp.float32),
            jax.ShapeDtypeStruct((n, 1), jnp.float32),
            jax.ShapeDtypeStruct((2 * n, 1), jnp.int32),
            jax.ShapeDtypeStruct((_LANE, 1), jnp.int32),
            jax.ShapeDtypeStruct((1, 1), jnp.float32),
        ),
        in_specs=[
            pl.BlockSpec((n, d), lambda: (0, 0)),
            pl.BlockSpec((d, _LANE), lambda: (0, 0)),
        ],
        out_specs=(
            pl.BlockSpec((n, 1), lambda: (0, 0)),
            pl.BlockSpec((n, 1), lambda: (0, 0)),
            pl.BlockSpec((2 * n, 1), lambda: (0, 0)),
            pl.BlockSpec((_LANE, 1), lambda: (0, 0)),
            pl.BlockSpec(memory_space=pltpu.SMEM),
        ),
        scratch_shapes=[pltpu.VMEM((2 * n, _LANE), jnp.float32)],
    )(x, wg_pad)


def _dispatch_call(pos_d, x):
    mesh = plsc.VectorSubcoreMesh(core_axis_name="c", subcore_axis_name="s")

    @functools.partial(
        pl.kernel, mesh=mesh,
        out_type=jax.ShapeDtypeStruct((_CAP, _D), jnp.float32),
        scratch_types=[
            pltpu.VMEM((2, _TPW), jnp.int32),
            pltpu.VMEM((_TPW, _D), jnp.float32),
            pltpu.SemaphoreType.DMA,
            pltpu.SemaphoreType.DMA,
        ],
    )
    def k(pos_hbm, x_hbm, xs_hbm, idx_v, rows_v, sem0, sem1):
        wid = lax.axis_index("c") * 16 + lax.axis_index("s")
        t0 = wid * _TPW
        pltpu.sync_copy(pos_hbm.at[wid], idx_v)
        pltpu.sync_copy(x_hbm.at[pl.ds(t0, _TPW), :], rows_v)
        c0 = pltpu.async_copy(rows_v, xs_hbm.at[idx_v.at[0]], sem0)
        c1 = pltpu.async_copy(rows_v, xs_hbm.at[idx_v.at[1]], sem1)
        c0.wait()
        c1.wait()

    return k(pos_d, x)


def _expert_body(be_ref, xs_ref, w1_ref, b1_ref, w2_ref, b2_ref, o_ref):
    h = jnp.dot(xs_ref[...], w1_ref[0],
                preferred_element_type=jnp.float32) + b1_ref[0]
    h = jnp.maximum(h, 0.0)
    out = jnp.dot(h, w2_ref[0], preferred_element_type=jnp.float32) + b2_ref[0]
    o_ref[...] = jnp.exp(out)


def _expert_call(x_sorted, blk_expert, W1, b1r, W2, b2r):
    grid_spec = pltpu.PrefetchScalarGridSpec(
        num_scalar_prefetch=1,
        grid=(_NBLK,),
        in_specs=[
            pl.BlockSpec((_B, _D), lambda i, be: (i, 0)),
            pl.BlockSpec((1, _D, _DFF), lambda i, be: (be[i], 0, 0)),
            pl.BlockSpec((1, 1, _DFF), lambda i, be: (be[i], 0, 0)),
            pl.BlockSpec((1, _DFF, _D), lambda i, be: (be[i], 0, 0)),
            pl.BlockSpec((1, 1, _D), lambda i, be: (be[i], 0, 0)),
        ],
        out_specs=pl.BlockSpec((_B, _D), lambda i, be: (i, 0)),
    )
    return pl.pallas_call(
        _expert_body,
        grid_spec=grid_spec,
        out_shape=jax.ShapeDtypeStruct((_CAP, _D), jnp.float32),
    )(blk_expert, x_sorted, W1, b1r, W2, b2r)


def _combine_gather_call(pos_r, out_exp):
    mesh = plsc.VectorSubcoreMesh(core_axis_name="c", subcore_axis_name="s")
    half = _TPW // 2

    @functools.partial(
        pl.kernel, mesh=mesh,
        out_type=(jax.ShapeDtypeStruct((_N, _D), jnp.float32),
                  jax.ShapeDtypeStruct((_N, _D), jnp.float32)),
        scratch_types=[
            pltpu.VMEM((2, 2, half), jnp.int32),
            pltpu.VMEM((2, half, _D), jnp.float32),
            pltpu.SemaphoreType.DMA,
            pltpu.SemaphoreType.DMA,
        ],
    )
    def k(pos_hbm, oe_hbm, c0_hbm, c1_hbm, idx_v, rows_v, sem0, sem1):
        wid = lax.axis_index("c") * 16 + lax.axis_index("s")
        t0 = wid * _TPW
        pltpu.sync_copy(pos_hbm.at[wid], idx_v)
        sems = (sem0, sem1)
        dsts = (c0_hbm, c0_hbm, c1_hbm, c1_hbm)
        cur = pltpu.async_copy(oe_hbm.at[idx_v.at[0, 0]], rows_v.at[0], sem0)
        for j in range(4):
            k_, h = j // 2, j % 2
            cur.wait()
            if j + 1 < 4:
                nxt = pltpu.async_copy(
                    oe_hbm.at[idx_v.at[(j + 1) // 2, (j + 1) % 2]],
                    rows_v.at[(j + 1) % 2], sems[(j + 1) % 2])
            pltpu.sync_copy(rows_v.at[j % 2],
                            dsts[j].at[pl.ds(t0 + h * half, half), :])
            if j + 1 < 4:
                cur = nxt

    return k(pos_r, out_exp)


def _combine_body(c0_ref, c1_ref, g1_ref, g2_ref, y_ref):
    bf = jnp.bfloat16
    g1 = g1_ref[...].astype(bf).astype(jnp.float32)
    g2 = g2_ref[...].astype(bf).astype(jnp.float32)
    c0 = c0_ref[...].astype(bf).astype(jnp.float32)
    c1 = c1_ref[...].astype(bf).astype(jnp.float32)
    comb = g1 * c0 + g2 * c1
    y_ref[...] = jnp.log(jnp.where(comb == 0.0, _EPS, comb))


def _combine_call(c0, c1, g1, g2):
    bn = 256
    return pl.pallas_call(
        _combine_body,
        grid=(_N // bn,),
        in_specs=[
            pl.BlockSpec((bn, _D), lambda i: (i, 0)),
            pl.BlockSpec((bn, _D), lambda i: (i, 0)),
            pl.BlockSpec((bn, 1), lambda i: (i, 0)),
            pl.BlockSpec((bn, 1), lambda i: (i, 0)),
        ],
        out_specs=pl.BlockSpec((bn, _D), lambda i: (i, 0)),
        out_shape=jax.ShapeDtypeStruct((_N, _D), jnp.float32),
    )(c0, c1, g1, g2)


def kernel(x, w_gate, w_noise, W1, b1, W2, b2):
    del w_noise
    wg_pad = jnp.pad(w_gate, ((0, 0), (0, _LANE - _E)))
    g1, g2, pos, blk_expert, loss = _gating_call(x, wg_pad)
    pos_r = pos.reshape(2, _NW, 2, _TPW // 2).transpose(1, 0, 2, 3)
    blk = blk_expert.reshape(_LANE)[:_NBLK]
    x_sorted = _dispatch_call(pos_r.reshape(_NW, 2, _TPW), x)
    out_exp = _expert_call(x_sorted, blk, W1, b1[:, None, :], W2,
                           b2[:, None, :])
    c0, c1 = _combine_gather_call(pos_r, out_exp)
    y = _combine_call(c0, c1, g1, g2)
    return y, loss[0, 0]


# speedup vs baseline: 2.2367x; 1.0022x over previous
import functools

import jax
import jax.numpy as jnp
import numpy as np
from jax import lax
from jax.experimental import pallas as pl
from jax.experimental.pallas import tpu as pltpu
from jax.experimental.pallas import tpu_sc as plsc

_N = 2048
_D = 1024
_E = 8
_DFF = 1024
_LANE = 128
_NEG = -3.0e38
_EPS = float(np.finfo(np.float64).eps)

_B = 256
_NBLK = (2 * _N) // _B + _E
_CAP = _NBLK * _B
_NW = 32
_TPW = _N // _NW


def _gating_body(x_ref, wg_ref, g1_ref, g2_ref, pos_ref, blk_ref,
                 loss_ref, o_ref):
    x = x_ref[...]
    logits = jnp.dot(x, wg_ref[...], preferred_element_type=jnp.float32)
    n = logits.shape[0]
    col = jax.lax.broadcasted_iota(jnp.int32, (n, _LANE), 1)
    valid = col < _E
    neg = jnp.where(valid, logits, _NEG)
    m1 = jnp.max(neg, axis=1, keepdims=True)
    i1 = jnp.min(jnp.where(neg == m1, col, _LANE), axis=1, keepdims=True)
    neg2 = jnp.where(col == i1, _NEG, neg)
    m2 = jnp.max(neg2, axis=1, keepdims=True)
    i2 = jnp.min(jnp.where(neg2 == m2, col, _LANE), axis=1, keepdims=True)
    t = jnp.exp(m2 - m1)
    g1 = 1.0 / (1.0 + t)
    g2 = t / (1.0 + t)
    g1_ref[...] = g1
    g2_ref[...] = g2

    one_hot1 = jnp.where(col == i1, 1.0, 0.0)
    one_hot2 = jnp.where(col == i2, 1.0, 0.0)
    o_ref[:n, :] = one_hot1
    o_ref[n:, :] = one_hot2

    gates = one_hot1 * g1 + one_hot2 * g2
    imp = jnp.sum(gates, axis=0, keepdims=True)
    load = jnp.sum((gates > 0.0).astype(jnp.float32), axis=0, keepdims=True)

    def _cv2(v):
        mean = jnp.sum(jnp.where(col[:1] < _E, v, 0.0)) / _E
        var = jnp.sum(jnp.where(col[:1] < _E, (v - mean) ** 2, 0.0)) / (_E - 1)
        return var / (mean * mean + 1e-10)

    loss_ref[0, 0] = _cv2(imp) + _cv2(load)

    counts = jnp.sum(one_hot1, axis=0, keepdims=True) + jnp.sum(
        one_hot2, axis=0, keepdims=True)
    pblocks = jnp.floor((counts + (_B - 1.0)) / _B)
    rowi = jax.lax.broadcasted_iota(jnp.int32, (_LANE, _LANE), 0)
    coli = jax.lax.broadcasted_iota(jnp.int32, (_LANE, _LANE), 1)
    tri_excl = jnp.where(rowi < coli, 1.0, 0.0)
    tri_incl = jnp.where(rowi <= coli, 1.0, 0.0)
    starts = jnp.dot(pblocks * _B, tri_excl,
                     preferred_element_type=jnp.float32)
    ecum_b = jnp.dot(pblocks, tri_incl,
                     preferred_element_type=jnp.float32)

    sub = rowi.astype(jnp.float32)
    hit = jnp.where((jnp.broadcast_to(ecum_b, (_LANE, _LANE)) <= sub)
                    & (coli < _E), 1.0, 0.0)
    blk_ref[...] = jnp.minimum(jnp.sum(hit, axis=1, keepdims=True),
                               _E - 1.0).astype(jnp.int32)

    r256 = jax.lax.broadcasted_iota(jnp.int32, (_B, _B), 0)
    c256 = jax.lax.broadcasted_iota(jnp.int32, (_B, _B), 1)
    tri256 = jnp.where(r256 >= c256, 1.0, 0.0)

    def _chunk(j, carry):
        blk = o_ref[pl.ds(j * _B, _B), :]
        incl = jnp.dot(tri256, blk, preferred_element_type=jnp.float32)
        posm = incl - blk + carry + starts
        posr = jnp.sum(jnp.where(blk > 0.0, posm, 0.0), axis=1, keepdims=True)
        pos_ref[pl.ds(j * _B, _B), :] = posr.astype(jnp.int32)
        return carry + jnp.sum(blk, axis=0, keepdims=True)

    lax.fori_loop(0, (2 * n) // _B, _chunk, jnp.zeros((1, _LANE), jnp.float32))


def _gating_call(x, wg_pad):
    n, d = x.shape
    return pl.pallas_call(
        _gating_body,
        out_shape=(
            jax.ShapeDtypeStruct((n, 1), jnp.float32),
            jax.ShapeDtypeStruct((n, 1), jnp.float32),
            jax.ShapeDtypeStruct((2 * n, 1), jnp.int32),
            jax.ShapeDtypeStruct((_LANE, 1), jnp.int32),
            jax.ShapeDtypeStruct((1, 1), jnp.float32),
        ),
        in_specs=[
            pl.BlockSpec((n, d), lambda: (0, 0)),
            pl.BlockSpec((d, _LANE), lambda: (0, 0)),
        ],
        out_specs=(
            pl.BlockSpec((n, 1), lambda: (0, 0)),
            pl.BlockSpec((n, 1), lambda: (0, 0)),
            pl.BlockSpec((2 * n, 1), lambda: (0, 0)),
            pl.BlockSpec((_LANE, 1), lambda: (0, 0)),
            pl.BlockSpec(memory_space=pltpu.SMEM),
        ),
        scratch_shapes=[pltpu.VMEM((2 * n, _LANE), jnp.float32)],
    )(x, wg_pad)


def _dispatch_call(pos_d, x):
    mesh = plsc.VectorSubcoreMesh(core_axis_name="c", subcore_axis_name="s")

    @functools.partial(
        pl.kernel, mesh=mesh,
        out_type=jax.ShapeDtypeStruct((_CAP, _D), jnp.float32),
        scratch_types=[
            pltpu.VMEM((2, _TPW), jnp.int32),
            pltpu.VMEM((_TPW, _D), jnp.float32),
            pltpu.SemaphoreType.DMA,
            pltpu.SemaphoreType.DMA,
        ],
    )
    def k(pos_hbm, x_hbm, xs_hbm, idx_v, rows_v, sem0, sem1):
        wid = lax.axis_index("c") * 16 + lax.axis_index("s")
        t0 = wid * _TPW
        pltpu.sync_copy(pos_hbm.at[wid], idx_v)
        pltpu.sync_copy(x_hbm.at[pl.ds(t0, _TPW), :], rows_v)
        c0 = pltpu.async_copy(rows_v, xs_hbm.at[idx_v.at[0]], sem0)
        c1 = pltpu.async_copy(rows_v, xs_hbm.at[idx_v.at[1]], sem1)
        c0.wait()
        c1.wait()

    return k(pos_d, x)


def _expert_body(be_ref, xs_ref, w1_ref, b1_ref, w2_ref, b2_ref, o_ref):
    h = jnp.dot(xs_ref[...], w1_ref[0],
                preferred_element_type=jnp.float32) + b1_ref[0]
    h = jnp.maximum(h, 0.0)
    out = jnp.dot(h, w2_ref[0], preferred_element_type=jnp.float32) + b2_ref[0]
    o_ref[...] = jnp.exp(out)


def _expert_call(x_sorted, blk_expert, W1, b1r, W2, b2r):
    grid_spec = pltpu.PrefetchScalarGridSpec(
        num_scalar_prefetch=1,
        grid=(_NBLK,),
        in_specs=[
            pl.BlockSpec((_B, _D), lambda i, be: (i, 0)),
            pl.BlockSpec((1, _D, _DFF), lambda i, be: (be[i], 0, 0)),
            pl.BlockSpec((1, 1, _DFF), lambda i, be: (be[i], 0, 0)),
            pl.BlockSpec((1, _DFF, _D), lambda i, be: (be[i], 0, 0)),
            pl.BlockSpec((1, 1, _D), lambda i, be: (be[i], 0, 0)),
        ],
        out_specs=pl.BlockSpec((_B, _D), lambda i, be: (i, 0)),
    )
    return pl.pallas_call(
        _expert_body,
        grid_spec=grid_spec,
        out_shape=jax.ShapeDtypeStruct((_CAP, _D), jnp.float32),
    )(blk_expert, x_sorted, W1, b1r, W2, b2r)


def _combine_gather_call(pos_r, out_exp):
    mesh = plsc.VectorSubcoreMesh(core_axis_name="c", subcore_axis_name="s")
    half = _TPW // 2

    @functools.partial(
        pl.kernel, mesh=mesh,
        out_type=(jax.ShapeDtypeStruct((_N, _D), jnp.float32),
                  jax.ShapeDtypeStruct((_N, _D), jnp.float32)),
        scratch_types=[
            pltpu.VMEM((2, 2, half), jnp.int32),
            pltpu.VMEM((2, half, _D), jnp.float32),
            pltpu.SemaphoreType.DMA,
            pltpu.SemaphoreType.DMA,
        ],
    )
    def k(pos_hbm, oe_hbm, c0_hbm, c1_hbm, idx_v, rows_v, sem0, sem1):
        wid = lax.axis_index("c") * 16 + lax.axis_index("s")
        t0 = wid * _TPW
        pltpu.sync_copy(pos_hbm.at[wid], idx_v)
        sems = (sem0, sem1)
        dsts = (c0_hbm, c0_hbm, c1_hbm, c1_hbm)
        cur = pltpu.async_copy(oe_hbm.at[idx_v.at[0, 0]], rows_v.at[0], sem0)
        for j in range(4):
            k_, h = j // 2, j % 2
            cur.wait()
            if j + 1 < 4:
                nxt = pltpu.async_copy(
                    oe_hbm.at[idx_v.at[(j + 1) // 2, (j + 1) % 2]],
                    rows_v.at[(j + 1) % 2], sems[(j + 1) % 2])
            pltpu.sync_copy(rows_v.at[j % 2],
                            dsts[j].at[pl.ds(t0 + h * half, half), :])
            if j + 1 < 4:
                cur = nxt

    return k(pos_r, out_exp)


def _combine_body(c0_ref, c1_ref, g1_ref, g2_ref, y_ref):
    bf = jnp.bfloat16
    g1 = g1_ref[...].astype(bf).astype(jnp.float32)
    g2 = g2_ref[...].astype(bf).astype(jnp.float32)
    c0 = c0_ref[...].astype(bf).astype(jnp.float32)
    c1 = c1_ref[...].astype(bf).astype(jnp.float32)
    comb = g1 * c0 + g2 * c1
    y_ref[...] = jnp.log(jnp.where(comb == 0.0, _EPS, comb))


def _combine_call(c0, c1, g1, g2):
    bn = 256
    return pl.pallas_call(
        _combine_body,
        grid=(_N // bn,),
        in_specs=[
            pl.BlockSpec((bn, _D), lambda i: (i, 0)),
            pl.BlockSpec((bn, _D), lambda i: (i, 0)),
            pl.BlockSpec((bn, 1), lambda i: (i, 0)),
            pl.BlockSpec((bn, 1), lambda i: (i, 0)),
        ],
        out_specs=pl.BlockSpec((bn, _D), lambda i: (i, 0)),
        out_shape=jax.ShapeDtypeStruct((_N, _D), jnp.float32),
    )(c0, c1, g1, g2)


def kernel(x, w_gate, w_noise, W1, b1, W2, b2):
    del w_noise
    wg_pad = jnp.pad(w_gate, ((0, 0), (0, _LANE - _E)))
    g1, g2, pos, blk_expert, loss = _gating_call(x, wg_pad)
    pos_r = pos.reshape(2, _NW, 2, _TPW // 2).transpose(1, 0, 2, 3)
    blk = blk_expert.reshape(_LANE)[:_NBLK]
    x_sorted = _dispatch_call(pos_r.reshape(_NW, 2, _TPW), x)
    out_exp = _expert_call(x_sorted, blk, W1, b1[:, None, :], W2,
                           b2[:, None, :])
    c0, c1 = _combine_gather_call(pos_r, out_exp)
    y = _combine_call(c0, c1, g1, g2)
    return y, loss[0, 0]
